# baseline (speedup 1.0000x reference)
import jax
import jax.numpy as jnp
from jax import lax
from jax.experimental import pallas as pl
from jax.experimental.pallas import tpu as pltpu
from jax.experimental.pallas import tpu_sc as plsc

_B = 16384
_D = 10
_DP = 16
_C1 = 1
_C2 = 3
_V = 1000000
_SR = 128
_NSR = _V * _DP // _SR
_L = 16

_NC = 2
_NS = 16
_NW = _NC * _NS
_BPW = _B // _NW
_CHUNK = 128
_NCHUNK = _BPW // _CHUNK


def _sc_gather_body(t1_hbm, t2_hbm, table_hbm, o1_hbm, o2_hbm,
                    idx1_v, idx2_v, q_v, off_v, sr_a, sr_b, out_v, sem):
  wid = lax.axis_index("s") * _NC + lax.axis_index("c")
  base = wid * _BPW
  pltpu.sync_copy(t1_hbm.at[pl.ds(base, _BPW)], idx1_v)
  pltpu.sync_copy(t2_hbm.at[pl.ds(base, _BPW)], idx2_v)
  bufs = [sr_a, sr_b]

  for tok_v, o_hbm in [(idx1_v, o1_hbm), (idx2_v, o2_hbm)]:
    for k in range(_BPW // _L):
      sl = pl.ds(k * _L, _L)
      tok = tok_v[sl]
      q_v[sl] = jnp.right_shift(tok, 3)
      off_v[sl] = (tok & 7) * _DP

    copies = [None] * _NCHUNK
    copies[0] = pltpu.async_copy(
        table_hbm.at[q_v.at[pl.ds(0, _CHUNK)]], bufs[0], sem)
    for c in range(_NCHUNK):
      if c + 1 < _NCHUNK:
        copies[c + 1] = pltpu.async_copy(
            table_hbm.at[q_v.at[pl.ds((c + 1) * _CHUNK, _CHUNK)]],
            bufs[(c + 1) % 2], sem)
      copies[c].wait()
      buf = bufs[c % 2]

      def body(j, _, c=c, buf=buf):
        g = c * _CHUNK + j
        gv = jnp.full((_L,), g, jnp.int32)
        offs = plsc.load_gather(off_v, [gv])
        colv = offs + lax.iota(jnp.int32, _L)
        vals = plsc.load_gather(buf, [jnp.full((_L,), j, jnp.int32), colv])
        out_v[pl.ds(g * _DP, _DP)] = vals
        return 0
      lax.fori_loop(0, _CHUNK, body, 0, unroll=4)
    pltpu.sync_copy(out_v, o_hbm.at[pl.ds(base * _DP, _BPW * _DP)])


def _sc_gather(t1, t2, table_sr):
  mesh = plsc.VectorSubcoreMesh(core_axis_name="c", subcore_axis_name="s")
  fn = pl.kernel(
      _sc_gather_body,
      out_type=[
          jax.ShapeDtypeStruct((_B * _DP,), jnp.float32),
          jax.ShapeDtypeStruct((_B * _DP,), jnp.float32),
      ],
      mesh=mesh,
      scratch_types=[
          pltpu.VMEM((_BPW,), jnp.int32),
          pltpu.VMEM((_BPW,), jnp.int32),
          pltpu.VMEM((_BPW,), jnp.int32),
          pltpu.VMEM((_BPW,), jnp.int32),
          pltpu.VMEM((_CHUNK, _SR), jnp.float32),
          pltpu.VMEM((_CHUNK, _SR), jnp.float32),
          pltpu.VMEM((_BPW * _DP,), jnp.float32),
          pltpu.SemaphoreType.DMA,
      ],
      compiler_params=pltpu.CompilerParams(
          use_tc_tiling_on_sc=True, needs_layout_passes=False),
  )
  return fn(t1, t2, table_sr)


def _dense_body(e1p_ref, e2p_ref, num_ref, w1a_ref, w1b_ref, w1c_ref,
                b1_ref, w2_ref, b2_ref, out_ref):
  e1p = e1p_ref[:]
  e2p = e2p_ref[:]
  ws1 = e1p[0:_B] + e1p[1:_B + 1] + e1p[2:_B + 2]
  ws2 = e2p[0:_B]
  for o in range(1, 2 * _C2 + 1):
    ws2 = ws2 + e2p[o:o + _B]
  i = lax.broadcasted_iota(jnp.int32, (_B, 1), 0)
  d1 = (1 + jnp.minimum(i, _C1) + jnp.minimum(_B - 1 - i, _C1)).astype(jnp.float32)
  d2 = (1 + jnp.minimum(i, _C2) + jnp.minimum(_B - 1 - i, _C2)).astype(jnp.float32)
  e1m = ws1 / d1
  e2m = ws2 / d2
  h = (jnp.dot(e1m, w1a_ref[:], preferred_element_type=jnp.float32)
       + jnp.dot(e2m, w1b_ref[:], preferred_element_type=jnp.float32)
       + jnp.dot(num_ref[:], w1c_ref[:], preferred_element_type=jnp.float32)
       + b1_ref[:])
  h = jnp.maximum(h, 0.0)
  logits = jnp.dot(h, w2_ref[:], preferred_element_type=jnp.float32) + b2_ref[:]
  out_ref[:] = jax.nn.sigmoid(logits)


def _dense(e1p, e2p, numeric, w1a, w1b, w1c, b1, W2, b2, interpret=False):
  return pl.pallas_call(
      _dense_body,
      out_shape=jax.ShapeDtypeStruct((_B, 1), jnp.float32),
      interpret=interpret,
  )(e1p, e2p, numeric, w1a, w1b, w1c, b1.reshape(1, -1), W2, b2.reshape(1, 1))


def kernel(text_1, text_2, numeric_features, table, W1, b1, W2, b2):
  t1 = text_1.astype(jnp.int32)
  t2 = text_2.astype(jnp.int32)
  zeros_c = jnp.zeros((_NSR, _DP - _D), table.dtype)
  parts = []
  for k in range(8):
    parts.append(lax.slice(table, (k, 0), (_V - 7 + k, _D), (8, 1)))
    parts.append(zeros_c)
  table_sr = jnp.concatenate(parts, axis=1)
  g1f, g2f = _sc_gather(t1, t2, table_sr)
  g1 = g1f.reshape(_B, _DP)
  g2 = g2f.reshape(_B, _DP)
  e1p = jnp.pad(g1, ((_C1, _C1), (0, 0)))
  e2p = jnp.pad(g2, ((_C2, _C2), (0, 0)))
  w1a = jnp.pad(W1[0:_D], ((0, _DP - _D), (0, 0)))
  w1b = jnp.pad(W1[_D:2 * _D], ((0, _DP - _D), (0, 0)))
  w1c = W1[2 * _D:]
  return _dense(e1p, e2p, numeric_features, w1a, w1b, w1c, b1, W2, b2)


# speedup vs baseline: 5.0589x; 5.0589x over previous
import jax
import jax.numpy as jnp
from jax import lax
from jax.experimental import pallas as pl
from jax.experimental.pallas import tpu as pltpu
from jax.experimental.pallas import tpu_sc as plsc

_B = 16384
_D = 10
_DP = 16
_C1 = 1
_C2 = 3
_V = 1000000
_SR = 128
_NSR = _V * _D // _SR
_L = 16

_NC = 2
_NS = 16
_NW = _NC * _NS
_BPW = _B // _NW
_CHUNK = 128
_NCHUNK = _BPW // _CHUNK


def _sc_gather_body(t1_hbm, t2_hbm, table_hbm, o1_hbm, o2_hbm,
                    idx1_v, idx2_v, q_v, qp_v, off_v, sr_a, sr_b, out_v, sem):
  wid = lax.axis_index("s") * _NC + lax.axis_index("c")
  base = wid * _BPW
  tbl = table_hbm
  pltpu.sync_copy(t1_hbm.at[pl.ds(base, _BPW)], idx1_v)
  pltpu.sync_copy(t2_hbm.at[pl.ds(base, _BPW)], idx2_v)
  bufs = [sr_a, sr_b]

  for tok_v, o_hbm in [(idx1_v, o1_hbm), (idx2_v, o2_hbm)]:
    for k in range(_BPW // _L):
      sl = pl.ds(k * _L, _L)
      flat = tok_v[sl] * _D
      q_v[sl] = jnp.right_shift(flat, 7)
      qp_v[sl] = jnp.minimum(jnp.right_shift(flat, 7) + 1, _NSR - 1)
      off_v[sl] = flat & (_SR - 1)

    copies = [None] * _NCHUNK
    copies[0] = [
        pltpu.async_copy(tbl.at[q_v.at[pl.ds(0, _CHUNK)]],
                         bufs[0].at[pl.ds(0, _CHUNK)], sem),
        pltpu.async_copy(tbl.at[qp_v.at[pl.ds(0, _CHUNK)]],
                         bufs[0].at[pl.ds(_CHUNK, _CHUNK)], sem),
    ]
    for c in range(_NCHUNK):
      if c + 1 < _NCHUNK:
        sl = pl.ds((c + 1) * _CHUNK, _CHUNK)
        copies[c + 1] = [
            pltpu.async_copy(tbl.at[q_v.at[sl]],
                             bufs[(c + 1) % 2].at[pl.ds(0, _CHUNK)], sem),
            pltpu.async_copy(tbl.at[qp_v.at[sl]],
                             bufs[(c + 1) % 2].at[pl.ds(_CHUNK, _CHUNK)], sem),
        ]
      for cp in copies[c]:
        cp.wait()
      buf = bufs[c % 2]

      def body(j, _, c=c, buf=buf):
        g = c * _CHUNK + j
        gv = jnp.full((_L,), g, jnp.int32)
        offs = plsc.load_gather(off_v, [gv])
        colv = offs + lax.iota(jnp.int32, _L)
        jv = jnp.full((_L,), j, jnp.int32)
        rowv = jnp.where(colv < _SR, jv, jv + _CHUNK)
        vals = plsc.load_gather(buf, [rowv, colv & (_SR - 1)])
        out_v[pl.ds(g * _DP, _DP)] = vals
        return 0
      lax.fori_loop(0, _CHUNK, body, 0, unroll=4)
    pltpu.sync_copy(out_v, o_hbm.at[pl.ds(base * _DP, _BPW * _DP)])


def _sc_gather(t1, t2, table_sr):
  mesh = plsc.VectorSubcoreMesh(core_axis_name="c", subcore_axis_name="s")
  fn = pl.kernel(
      _sc_gather_body,
      out_type=[
          jax.ShapeDtypeStruct((_B * _DP,), jnp.float32),
          jax.ShapeDtypeStruct((_B * _DP,), jnp.float32),
      ],
      mesh=mesh,
      scratch_types=[
          pltpu.VMEM((_BPW,), jnp.int32),
          pltpu.VMEM((_BPW,), jnp.int32),
          pltpu.VMEM((_BPW,), jnp.int32),
          pltpu.VMEM((_BPW,), jnp.int32),
          pltpu.VMEM((_BPW,), jnp.int32),
          pltpu.VMEM((2 * _CHUNK, _SR), jnp.float32),
          pltpu.VMEM((2 * _CHUNK, _SR), jnp.float32),
          pltpu.VMEM((_BPW * _DP,), jnp.float32),
          pltpu.SemaphoreType.DMA,
      ],
      compiler_params=pltpu.CompilerParams(
          use_tc_tiling_on_sc=False, needs_layout_passes=False),
  )
  return fn(t1, t2, table_sr)


def _dense_body(e1p_ref, e2p_ref, num_ref, w1a_ref, w1b_ref, w1c_ref,
                b1_ref, w2_ref, b2_ref, out_ref):
  e1p = e1p_ref[:]
  e2p = e2p_ref[:]
  ws1 = e1p[0:_B] + e1p[1:_B + 1] + e1p[2:_B + 2]
  ws2 = e2p[0:_B]
  for o in range(1, 2 * _C2 + 1):
    ws2 = ws2 + e2p[o:o + _B]
  i = lax.broadcasted_iota(jnp.int32, (_B, 1), 0)
  d1 = (1 + jnp.minimum(i, _C1) + jnp.minimum(_B - 1 - i, _C1)).astype(jnp.float32)
  d2 = (1 + jnp.minimum(i, _C2) + jnp.minimum(_B - 1 - i, _C2)).astype(jnp.float32)
  e1m = ws1 / d1
  e2m = ws2 / d2
  h = (jnp.dot(e1m, w1a_ref[:], preferred_element_type=jnp.float32)
       + jnp.dot(e2m, w1b_ref[:], preferred_element_type=jnp.float32)
       + jnp.dot(num_ref[:], w1c_ref[:], preferred_element_type=jnp.float32)
       + b1_ref[:])
  h = jnp.maximum(h, 0.0)
  logits = jnp.dot(h, w2_ref[:], preferred_element_type=jnp.float32) + b2_ref[:]
  out_ref[:] = jax.nn.sigmoid(logits)


def _dense(e1p, e2p, numeric, w1a, w1b, w1c, b1, W2, b2, interpret=False):
  return pl.pallas_call(
      _dense_body,
      out_shape=jax.ShapeDtypeStruct((_B, 1), jnp.float32),
      interpret=interpret,
  )(e1p, e2p, numeric, w1a, w1b, w1c, b1.reshape(1, -1), W2, b2.reshape(1, 1))


def kernel(text_1, text_2, numeric_features, table, W1, b1, W2, b2):
  t1 = text_1.astype(jnp.int32)
  t2 = text_2.astype(jnp.int32)
  g1f, g2f = _sc_gather(t1, t2, table.reshape(_NSR, _SR))
  g1 = g1f.reshape(_B, _DP)
  g2 = g2f.reshape(_B, _DP)
  e1p = jnp.pad(g1, ((_C1, _C1), (0, 0)))
  e2p = jnp.pad(g2, ((_C2, _C2), (0, 0)))
  w1a = jnp.pad(W1[0:_D], ((0, _DP - _D), (0, 0)))
  w1b = jnp.pad(W1[_D:2 * _D], ((0, _DP - _D), (0, 0)))
  w1c = W1[2 * _D:]
  return _dense(e1p, e2p, numeric_features, w1a, w1b, w1c, b1, W2, b2)
